# Initial kernel scaffold; baseline (speedup 1.0000x reference)
import jax
import jax.numpy as jnp
from jax.experimental import pallas as pl


def kernel(samples, Ws1, Wn1, b1, Ws2, Wn2, b2, Wp1, bp1, Wp2, bp2):
    raise NotImplementedError("write your pallas kernel here")


# speedup vs baseline: 696.4288x; 696.4288x over previous
import jax
import jax.numpy as jnp
from jax.experimental import pallas as pl


def _fused_kernel(x_ref, Ws1_ref, Wn1_ref, b1_ref, Ws2_ref, Wn2_ref, b2_ref,
                  Wp1_ref, bp1_ref, Wp2_ref, bp2_ref, loc_ref, glob_ref):
    x = x_ref[...]
    B, N, D = x.shape
    inv_deg = 1.0 / (N - 1)
    xf = x.reshape(B * N, D)

    Wn1 = Wn1_ref[...]
    t1 = jnp.dot(xf, Ws1_ref[...] - inv_deg * Wn1,
                 preferred_element_type=jnp.float32)
    s1 = jnp.sum(x, axis=1) * inv_deg
    c1 = jnp.dot(s1, Wn1, preferred_element_type=jnp.float32) + b1_ref[...]
    h = jnp.maximum(t1.reshape(B, N, -1) + c1[:, None, :], 0.0)

    Wn2 = Wn2_ref[...]
    t2 = jnp.dot(h.reshape(B * N, -1), Ws2_ref[...] - inv_deg * Wn2,
                 preferred_element_type=jnp.float32)
    s2 = jnp.sum(h, axis=1) * inv_deg
    c2 = jnp.dot(s2, Wn2, preferred_element_type=jnp.float32) + b2_ref[...]
    h2 = jnp.maximum(t2.reshape(B, N, -1) + c2[:, None, :], 0.0)

    p = jnp.maximum(jnp.dot(h2.reshape(B * N, -1), Wp1_ref[...],
                            preferred_element_type=jnp.float32)
                    + bp1_ref[...], 0.0)
    out = jnp.dot(p, Wp2_ref[...],
                  preferred_element_type=jnp.float32) + bp2_ref[...]
    out = out.reshape(B, N, -1)

    loc_ref[...] = out
    glob_ref[...] = jnp.mean(out, axis=1)


def kernel(samples, Ws1, Wn1, b1, Ws2, Wn2, b2, Wp1, bp1, Wp2, bp2):
    B, N, _ = samples.shape
    p_o = Wp2.shape[1]
    loc, glob = pl.pallas_call(
        _fused_kernel,
        out_shape=(
            jax.ShapeDtypeStruct((B, N, p_o), jnp.float32),
            jax.ShapeDtypeStruct((B, p_o), jnp.float32),
        ),
    )(samples, Ws1, Wn1, b1.reshape(1, -1), Ws2, Wn2, b2.reshape(1, -1),
      Wp1, bp1.reshape(1, -1), Wp2, bp2.reshape(1, -1))
    return glob[:, None, :], loc
